# baseline (speedup 1.0000x reference)
import jax
import jax.numpy as jnp
from jax import lax
from jax.experimental import pallas as pl
from jax.experimental.pallas import tpu as pltpu
from jax.experimental.pallas import tpu_sc as plsc

_B, _L, _H = 1024, 200, 64
_NC, _NS = 2, 16
_NW = _NC * _NS
_CH = 100
_TOK = _B * _L
_GCH = _TOK // _CH
_NCH = _GCH // _NW


def _emb_body(ids_hbm, tok_hbm, pos_hbm, out_hbm, idx_v, pos_v, buf_v, sem):
    wid = lax.axis_index("s") * _NC + lax.axis_index("c")
    pltpu.sync_copy(ids_hbm.at[pl.ds(wid * _NCH, _NCH)], idx_v)
    pltpu.sync_copy(pos_hbm.at[pl.ds(0, _L)], pos_v)

    def chunk_body(c, carry):
        pltpu.async_copy(tok_hbm.at[idx_v.at[c]], buf_v, sem).wait()
        base_p = (c % 2) * _CH

        def row_body(t, carry2):
            p = base_p + t
            for d in range(_H // 16):
                sl = pl.ds(d * 16, 16)
                buf_v[t, sl] = buf_v[t, sl] + pos_v[p, sl]
            return carry2

        lax.fori_loop(0, _CH, row_body, 0)
        pltpu.sync_copy(buf_v, out_hbm.at[wid * _NCH + c])
        return carry

    lax.fori_loop(0, _NCH, chunk_body, 0)


@jax.jit
def kernel(input_ids, token_table, pos_table):
    ids = input_ids.reshape(_GCH, _CH).astype(jnp.int32)
    out = pl.kernel(
        _emb_body,
        out_type=jax.ShapeDtypeStruct((_GCH, _CH, _H), jnp.float32),
        mesh=plsc.VectorSubcoreMesh(core_axis_name="c", subcore_axis_name="s"),
        compiler_params=pltpu.CompilerParams(use_tc_tiling_on_sc=False),
        scratch_types=[
            pltpu.VMEM((_NCH, _CH), jnp.int32),
            pltpu.VMEM((_L, _H), jnp.float32),
            pltpu.VMEM((_CH, _H), jnp.float32),
            pltpu.SemaphoreType.DMA,
        ],
    )(ids, token_table, pos_table)
    return out.reshape(_B, _L, _H)


# speedup vs baseline: 1.0567x; 1.0567x over previous
import jax
import jax.numpy as jnp
from jax import lax
from jax.experimental import pallas as pl
from jax.experimental.pallas import tpu as pltpu
from jax.experimental.pallas import tpu_sc as plsc

_B, _L, _H = 1024, 200, 64
_NC, _NS = 2, 16
_NW = _NC * _NS
_CH = 100
_TOK = _B * _L
_GCH = _TOK // _CH
_NCH = _GCH // _NW


def _emb_body(ids_hbm, tok_hbm, pos_hbm, out_hbm, idx_v, pos_v, buf_v, gsem, osem):
    wid = lax.axis_index("s") * _NC + lax.axis_index("c")
    base = wid * _NCH
    pltpu.sync_copy(ids_hbm.at[pl.ds(base, _NCH)], idx_v)
    pltpu.sync_copy(pos_hbm.at[pl.ds(0, _L)], pos_v)

    pltpu.async_copy(tok_hbm.at[idx_v.at[0]], buf_v.at[0], gsem)

    def chunk_body(c, carry):
        s = c % 2

        @pl.when(c >= 1)
        def _():
            pltpu.make_async_copy(buf_v.at[1 - s], out_hbm.at[base + c - 1], osem).wait()

        @pl.when(c + 1 < _NCH)
        def _():
            pltpu.async_copy(tok_hbm.at[idx_v.at[c + 1]], buf_v.at[1 - s], gsem)

        pltpu.make_async_copy(tok_hbm.at[idx_v.at[c]], buf_v.at[s], gsem).wait()

        base_p = (c % 2) * _CH

        def row_body(t, carry2):
            p = base_p + t
            for d in range(_H // 16):
                sl = pl.ds(d * 16, 16)
                buf_v[s, t, sl] = buf_v[s, t, sl] + pos_v[p, sl]
            return carry2

        lax.fori_loop(0, _CH, row_body, 0)

        pltpu.async_copy(buf_v.at[s], out_hbm.at[base + c], osem)
        return carry

    lax.fori_loop(0, _NCH, chunk_body, 0)
    pltpu.make_async_copy(buf_v.at[1], out_hbm.at[base + _NCH - 1], osem).wait()


@jax.jit
def kernel(input_ids, token_table, pos_table):
    ids = input_ids.reshape(_GCH, _CH).astype(jnp.int32)
    out = pl.kernel(
        _emb_body,
        out_type=jax.ShapeDtypeStruct((_GCH, _CH, _H), jnp.float32),
        mesh=plsc.VectorSubcoreMesh(core_axis_name="c", subcore_axis_name="s"),
        compiler_params=pltpu.CompilerParams(use_tc_tiling_on_sc=False),
        scratch_types=[
            pltpu.VMEM((_NCH, _CH), jnp.int32),
            pltpu.VMEM((_L, _H), jnp.float32),
            pltpu.VMEM((2, _CH, _H), jnp.float32),
            pltpu.SemaphoreType.DMA,
            pltpu.SemaphoreType.DMA,
        ],
    )(ids, token_table, pos_table)
    return out.reshape(_B, _L, _H)
